# baseline (speedup 1.0000x reference)
import functools
from math import ceil

import jax
import jax.numpy as jnp
from jax import lax
from jax.experimental import pallas as pl
from jax.experimental.pallas import tpu as pltpu
from jax.experimental.pallas import tpu_sc as plsc

_B, _N, _P = 4, 512, 16384
_PBLK = 2048
_LANES = 128
_NUM = ceil(0.9 * (_N - 1))
_K = (_N - 1) - _NUM
_L = 16
_NV = _N // _L


def _mv_body(dens_ref, prob_ref, out_ref, acc_ref):
    p = pl.program_id(1)

    @pl.when(p == 0)
    def _init():
        acc_ref[...] = jnp.zeros_like(acc_ref)

    blk = prob_ref[0]
    d = dens_ref[0]
    acc = acc_ref[...]
    for j in range(_PBLK // _LANES):
        sl = slice(j * _LANES, (j + 1) * _LANES)
        acc = acc + blk[:, sl] * d[:, sl]
    acc_ref[...] = acc

    @pl.when(p == pl.num_programs(1) - 1)
    def _fin():
        out_ref[...] = jnp.sum(acc_ref[...], axis=1)[None, None]


def _matvec(prob_list, pre_density):
    return pl.pallas_call(
        _mv_body,
        grid=(_B, _P // _PBLK),
        in_specs=[
            pl.BlockSpec((1, 1, _PBLK), lambda b, p: (b, 0, p)),
            pl.BlockSpec((1, _N, _PBLK), lambda b, p: (b, 0, p)),
        ],
        out_specs=pl.BlockSpec((1, 1, _N), lambda b, p: (b, 0, 0)),
        out_shape=jax.ShapeDtypeStruct((_B, 1, _N), jnp.float32),
        scratch_shapes=[pltpu.VMEM((_N, _LANES), jnp.float32)],
    )(pre_density.reshape(_B, 1, _P), prob_list).reshape(_B, _N)


def _gather16(v, idx):
    return lax.gather(
        v,
        idx[:, None],
        lax.GatherDimensionNumbers(
            offset_dims=(), collapsed_slice_dims=(0,), start_index_map=(0,)
        ),
        (1,),
        mode=lax.GatherScatterMode.PROMISE_IN_BOUNDS,
    )


def _xlane_sum(v):
    lane = lax.iota(jnp.int32, _L)
    for s in (1, 2, 4, 8):
        v = v + _gather16(v, lane ^ s)
    return v


def _sc_loss_body(pc_hbm, tp_hbm, out_hbm, pc_v, tp_v, res_v, out_v):
    cid = lax.axis_index("c")
    sid = lax.axis_index("s")
    wid = sid * 2 + cid

    @pl.when(wid < _B)
    def _work():
        b = wid
        pltpu.sync_copy(pc_hbm.at[b], pc_v)
        pltpu.sync_copy(tp_hbm.at[b], tp_v)

        lane = lax.iota(jnp.int32, _L)
        last = lane == (_L - 1)
        total_vec = jnp.zeros((_L,), jnp.float32)
        res511_vec = jnp.zeros((_L,), jnp.float32)
        for j in range(_NV):
            sl = pl.ds(j * _L, _L)
            r = jnp.abs(tp_v[sl] - pc_v[sl])
            if j == _NV - 1:
                res511_vec = jnp.where(last, r, 0.0)
                r = jnp.where(last, jnp.float32(-1.0), r)
                total_vec = total_vec + jnp.where(last, 0.0, r)
            else:
                total_vec = total_vec + r
            res_v[sl] = r
        total = _xlane_sum(total_vec)
        res511 = _xlane_sum(res511_vec)

        km1 = jnp.full((_L,), _K - 1, jnp.int32)

        def _bs_body(_, carry):
            lo, hi = carry
            mid = lo + lax.shift_right_logical(hi - lo, 1)
            t = plsc.bitcast(mid, jnp.float32)
            cnt_vec = jnp.zeros((_L,), jnp.int32)
            for j in range(_NV):
                r = res_v[pl.ds(j * _L, _L)]
                cnt_vec = cnt_vec + jnp.where(r > t, 1, 0).astype(jnp.int32)
            cnt = _xlane_sum(cnt_vec)
            pred = cnt > km1
            return (jnp.where(pred, mid + 1, lo), jnp.where(pred, hi, mid))

        lo, _ = lax.fori_loop(
            0,
            31,
            _bs_body,
            (
                jnp.zeros((_L,), jnp.int32),
                jnp.full((_L,), 0x7F800000, jnp.int32),
            ),
        )

        tvec = plsc.bitcast(lo, jnp.float32)
        sum_gt_vec = jnp.zeros((_L,), jnp.float32)
        cnt_gt_vec = jnp.zeros((_L,), jnp.int32)
        for j in range(_NV):
            r = res_v[pl.ds(j * _L, _L)]
            m = r > tvec
            sum_gt_vec = sum_gt_vec + jnp.where(m, r, 0.0)
            cnt_gt_vec = cnt_gt_vec + jnp.where(m, 1, 0).astype(jnp.int32)
        sum_gt = _xlane_sum(sum_gt_vec)
        cnt_gt = _xlane_sum(cnt_gt_vec)
        sum_top = sum_gt + (jnp.full((_L,), _K, jnp.int32) - cnt_gt).astype(
            jnp.float32
        ) * tvec

        out_v[...] = total - sum_top + res511
        pltpu.sync_copy(out_v, out_hbm.at[b])


@functools.cache
def _sc_loss():
    return pl.kernel(
        _sc_loss_body,
        out_type=jax.ShapeDtypeStruct((_B, _L), jnp.float32),
        mesh=plsc.VectorSubcoreMesh(
            core_axis_name="c", subcore_axis_name="s", num_cores=2, num_subcores=16
        ),
        compiler_params=pltpu.CompilerParams(needs_layout_passes=False),
        scratch_types=[
            pltpu.VMEM((_N,), jnp.float32),
            pltpu.VMEM((_N,), jnp.float32),
            pltpu.VMEM((_N,), jnp.float32),
            pltpu.VMEM((_L,), jnp.float32),
        ],
    )


def kernel(prob_list, target_list, pre_density):
    pre_count = _matvec(prob_list, pre_density)
    tpad = jnp.zeros((_B, _N), jnp.float32).at[:, : _N - 1].set(target_list)
    per_batch = _sc_loss()(pre_count, tpad)
    return jnp.sum(per_batch[:, 0]) / _B
